# baseline (speedup 1.0000x reference)
import functools

import jax
import jax.numpy as jnp
from jax.experimental import pallas as pl


def _loss_kernel(q_ref, items_ref, out_ref, *, block_t: int, m: int):
    i = pl.program_id(0)

    items = items_ref[...]
    n2 = jnp.sum(items * items, axis=1)

    q = q_ref[...]
    score = jax.lax.dot_general(
        q.astype(jnp.bfloat16), items.astype(jnp.bfloat16),
        (((1,), (1,)), ((), ())),
        preferred_element_type=jnp.float32,
    )

    rowmax = jnp.max(score, axis=1, keepdims=True)
    mask = (score == rowmax).astype(jnp.bfloat16)
    n2_and_ones = jnp.concatenate(
        [n2[:, None], jnp.ones((m, 1), jnp.float32)], axis=1
    ).astype(jnp.bfloat16)
    picked = jax.lax.dot_general(
        mask, n2_and_ones, (((1,), (0,)), ((), ())),
        preferred_element_type=jnp.float32,
    )
    n2_at = picked[:, 0] / picked[:, 1]

    partial = (
        jnp.sum(q * q)
        + jnp.sum(n2_at)
        - 2.0 * jnp.sum(rowmax)
    )

    @pl.when(i == 0)
    def _():
        out_ref[...] = jnp.zeros_like(out_ref)

    out_ref[...] += jnp.reshape(partial, (1, 1))


@jax.jit
def kernel(queries, items):
    n, l, c = queries.shape
    m = items.shape[0]
    t = n * l
    q = queries.reshape(t, c)

    block_t = 2048
    grid = (t // block_t,)

    total = pl.pallas_call(
        functools.partial(_loss_kernel, block_t=block_t, m=m),
        grid=grid,
        in_specs=[
            pl.BlockSpec((block_t, c), lambda i: (i, 0)),
            pl.BlockSpec((m, c), lambda i: (0, 0)),
        ],
        out_specs=pl.BlockSpec((1, 1), lambda i: (0, 0)),
        out_shape=jax.ShapeDtypeStruct((1, 1), jnp.float32),
    )(q, items)

    return total[0, 0] / (t * c)


# speedup vs baseline: 1.1031x; 1.1031x over previous
import functools

import jax
import jax.numpy as jnp
from jax.experimental import pallas as pl


def _loss_kernel(q_ref, items_ref, out_ref, *, block_t: int, m: int):
    i = pl.program_id(0)

    items = items_ref[...]
    n2 = jnp.sum(items * items, axis=1)

    q = q_ref[...]
    score = jax.lax.dot_general(
        q, items, (((1,), (1,)), ((), ())),
        preferred_element_type=jnp.float32,
    )

    rowmax = jnp.max(score, axis=1, keepdims=True)
    mask = (score == rowmax).astype(jnp.float32)
    n2_and_ones = jnp.concatenate(
        [n2[:, None], jnp.ones((m, 1), jnp.float32)], axis=1
    )
    picked = jax.lax.dot_general(
        mask, n2_and_ones, (((1,), (0,)), ((), ())),
        preferred_element_type=jnp.float32,
    )
    n2_at = picked[:, 0] / picked[:, 1]

    partial = (
        jnp.sum(q * q)
        + jnp.sum(n2_at)
        - 2.0 * jnp.sum(rowmax)
    )

    @pl.when(i == 0)
    def _():
        out_ref[...] = jnp.zeros_like(out_ref)

    out_ref[...] += jnp.reshape(partial, (1, 1))


@jax.jit
def kernel(queries, items):
    n, l, c = queries.shape
    m = items.shape[0]
    t = n * l
    q = queries.reshape(t, c)

    block_t = 4096
    grid = (t // block_t,)

    total = pl.pallas_call(
        functools.partial(_loss_kernel, block_t=block_t, m=m),
        grid=grid,
        in_specs=[
            pl.BlockSpec((block_t, c), lambda i: (i, 0)),
            pl.BlockSpec((m, c), lambda i: (0, 0)),
        ],
        out_specs=pl.BlockSpec((1, 1), lambda i: (0, 0)),
        out_shape=jax.ShapeDtypeStruct((1, 1), jnp.float32),
    )(q, items)

    return total[0, 0] / (t * c)


# speedup vs baseline: 1.1077x; 1.0042x over previous
import functools

import jax
import jax.numpy as jnp
from jax.experimental import pallas as pl


def _loss_kernel(q_ref, items_ref, out_ref, *, block_t: int, m: int):
    i = pl.program_id(0)

    items = items_ref[...]
    n2 = jnp.sum(items * items, axis=1)

    q = q_ref[...]
    score = jax.lax.dot_general(
        q, items, (((1,), (1,)), ((), ())),
        preferred_element_type=jnp.float32,
    )

    rowmax = jnp.max(score, axis=1, keepdims=True)
    mask = (score == rowmax).astype(jnp.float32)
    n2_and_ones = jnp.concatenate(
        [n2[:, None], jnp.ones((m, 1), jnp.float32)], axis=1
    )
    picked = jax.lax.dot_general(
        mask, n2_and_ones, (((1,), (0,)), ((), ())),
        preferred_element_type=jnp.float32,
    )
    n2_at = picked[:, 0] / picked[:, 1]

    partial = (
        jnp.sum(q * q)
        + jnp.sum(n2_at)
        - 2.0 * jnp.sum(rowmax)
    )

    @pl.when(i == 0)
    def _():
        out_ref[...] = jnp.zeros_like(out_ref)

    out_ref[...] += jnp.reshape(partial, (1, 1))


@jax.jit
def kernel(queries, items):
    n, l, c = queries.shape
    m = items.shape[0]
    t = n * l
    q = queries.reshape(t, c)

    block_t = 8192
    grid = (t // block_t,)

    total = pl.pallas_call(
        functools.partial(_loss_kernel, block_t=block_t, m=m),
        grid=grid,
        in_specs=[
            pl.BlockSpec((block_t, c), lambda i: (i, 0)),
            pl.BlockSpec((m, c), lambda i: (0, 0)),
        ],
        out_specs=pl.BlockSpec((1, 1), lambda i: (0, 0)),
        out_shape=jax.ShapeDtypeStruct((1, 1), jnp.float32),
    )(q, items)

    return total[0, 0] / (t * c)


# speedup vs baseline: 1.3018x; 1.1752x over previous
import functools

import jax
import jax.numpy as jnp
from jax.experimental import pallas as pl


def _loss_kernel(q_ref, items_ref, out_ref, *, block_t: int, chunk_t: int,
                 m: int):
    i = pl.program_id(0)

    items = items_ref[...]
    n2 = jnp.sum(items * items, axis=1)
    n2_and_ones = jnp.concatenate(
        [n2[:, None], jnp.ones((m, 1), jnp.float32)], axis=1
    )

    partial = jnp.zeros((), jnp.float32)
    for k in range(block_t // chunk_t):
        q = q_ref[pl.ds(k * chunk_t, chunk_t), :]
        score = jax.lax.dot_general(
            q, items, (((1,), (1,)), ((), ())),
            preferred_element_type=jnp.float32,
        )
        rowmax = jnp.max(score, axis=1, keepdims=True)
        mask = (score == rowmax).astype(jnp.float32)
        picked = jax.lax.dot_general(
            mask, n2_and_ones, (((1,), (0,)), ((), ())),
            preferred_element_type=jnp.float32,
        )
        n2_at = picked[:, 0] / picked[:, 1]
        partial += (
            jnp.sum(q * q)
            + jnp.sum(n2_at)
            - 2.0 * jnp.sum(rowmax)
        )

    @pl.when(i == 0)
    def _():
        out_ref[...] = jnp.zeros_like(out_ref)

    out_ref[...] += jnp.reshape(partial, (1, 1))


@jax.jit
def kernel(queries, items):
    n, l, c = queries.shape
    m = items.shape[0]
    t = n * l
    q = queries.reshape(t, c)

    block_t = 4096
    chunk_t = 2048
    grid = (t // block_t,)

    total = pl.pallas_call(
        functools.partial(_loss_kernel, block_t=block_t, chunk_t=chunk_t,
                          m=m),
        grid=grid,
        in_specs=[
            pl.BlockSpec((block_t, c), lambda i: (i, 0)),
            pl.BlockSpec((m, c), lambda i: (0, 0)),
        ],
        out_specs=pl.BlockSpec((1, 1), lambda i: (0, 0)),
        out_shape=jax.ShapeDtypeStruct((1, 1), jnp.float32),
    )(q, items)

    return total[0, 0] / (t * c)


# speedup vs baseline: 1.4524x; 1.1157x over previous
import functools

import jax
import jax.numpy as jnp
from jax.experimental import pallas as pl


def _loss_kernel(q_ref, items_ref, out_ref, *, block_t: int, chunk_t: int,
                 m: int):
    i = pl.program_id(0)

    items = items_ref[...]
    n2 = jnp.sum(items * items, axis=1)
    n2_and_ones = jnp.concatenate(
        [n2[:, None], jnp.ones((m, 1), jnp.float32)], axis=1
    )

    partial = jnp.zeros((), jnp.float32)
    for k in range(block_t // chunk_t):
        q = q_ref[pl.ds(k * chunk_t, chunk_t), :]
        score = jax.lax.dot_general(
            q, items, (((1,), (1,)), ((), ())),
            preferred_element_type=jnp.float32,
        )
        rowmax = jnp.max(score, axis=1, keepdims=True)
        mask = (score == rowmax).astype(jnp.float32)
        picked = jax.lax.dot_general(
            mask, n2_and_ones, (((1,), (0,)), ((), ())),
            preferred_element_type=jnp.float32,
        )
        n2_at = picked[:, 0] / picked[:, 1]
        partial += (
            jnp.sum(q * q)
            + jnp.sum(n2_at)
            - 2.0 * jnp.sum(rowmax)
        )

    @pl.when(i == 0)
    def _():
        out_ref[...] = jnp.zeros_like(out_ref)

    out_ref[...] += jnp.reshape(partial, (1, 1))


@jax.jit
def kernel(queries, items):
    n, l, c = queries.shape
    m = items.shape[0]
    t = n * l
    q = queries.reshape(t, c)

    block_t = 8192
    chunk_t = 2048
    grid = (t // block_t,)

    total = pl.pallas_call(
        functools.partial(_loss_kernel, block_t=block_t, chunk_t=chunk_t,
                          m=m),
        grid=grid,
        in_specs=[
            pl.BlockSpec((block_t, c), lambda i: (i, 0)),
            pl.BlockSpec((m, c), lambda i: (0, 0)),
        ],
        out_specs=pl.BlockSpec((1, 1), lambda i: (0, 0)),
        out_shape=jax.ShapeDtypeStruct((1, 1), jnp.float32),
    )(q, items)

    return total[0, 0] / (t * c)


# speedup vs baseline: 1.5311x; 1.0542x over previous
import functools

import jax
import jax.numpy as jnp
from jax.experimental import pallas as pl


def _loss_kernel(q_ref, items_ref, out_ref, *, block_t: int, chunk_t: int,
                 m: int):
    i = pl.program_id(0)

    items = items_ref[...]
    n2 = jnp.sum(items * items, axis=1)
    n2_and_ones = jnp.concatenate(
        [n2[:, None], jnp.ones((m, 1), jnp.float32)], axis=1
    )

    partial = jnp.zeros((), jnp.float32)
    for k in range(block_t // chunk_t):
        q = q_ref[pl.ds(k * chunk_t, chunk_t), :]
        score = jax.lax.dot_general(
            q, items, (((1,), (1,)), ((), ())),
            preferred_element_type=jnp.float32,
        )
        rowmax = jnp.max(score, axis=1, keepdims=True)
        mask = (score == rowmax).astype(jnp.float32)
        picked = jax.lax.dot_general(
            mask, n2_and_ones, (((1,), (0,)), ((), ())),
            preferred_element_type=jnp.float32,
        )
        n2_at = picked[:, 0] / picked[:, 1]
        partial += (
            jnp.sum(q * q)
            + jnp.sum(n2_at)
            - 2.0 * jnp.sum(rowmax)
        )

    @pl.when(i == 0)
    def _():
        out_ref[...] = jnp.zeros_like(out_ref)

    out_ref[...] += jnp.reshape(partial, (1, 1))


@jax.jit
def kernel(queries, items):
    n, l, c = queries.shape
    m = items.shape[0]
    t = n * l
    q = queries.reshape(t, c)

    block_t = 8192
    chunk_t = 1024
    grid = (t // block_t,)

    total = pl.pallas_call(
        functools.partial(_loss_kernel, block_t=block_t, chunk_t=chunk_t,
                          m=m),
        grid=grid,
        in_specs=[
            pl.BlockSpec((block_t, c), lambda i: (i, 0)),
            pl.BlockSpec((m, c), lambda i: (0, 0)),
        ],
        out_specs=pl.BlockSpec((1, 1), lambda i: (0, 0)),
        out_shape=jax.ShapeDtypeStruct((1, 1), jnp.float32),
    )(q, items)

    return total[0, 0] / (t * c)
